# baseline (speedup 1.0000x reference)
import jax
import jax.numpy as jnp
from jax.experimental import pallas as pl
from jax.experimental.pallas import tpu as pltpu

B = 32
V = 100000
D = 64
TAIL = (V // 128) * 128
VTAIL = V - TAIL
LAST_FULL = (V // 128) - 1


def _body(i_ref, wt_ref, out_ref, bufs_ref, tail_ref, sems):
    for b in range(B):
        cb = pl.multiple_of(jnp.minimum(i_ref[b] // 128, LAST_FULL) * 128, 128)
        pltpu.make_async_copy(
            wt_ref.at[:, pl.ds(cb, 128)], bufs_ref.at[b], sems.at[b]
        ).start()
    pltpu.make_async_copy(
        wt_ref.at[:, pl.ds(TAIL, VTAIL)], tail_ref, sems.at[B]
    ).start()
    lane128 = jax.lax.broadcasted_iota(jnp.int32, (D, 128), 1)
    lane32 = jax.lax.broadcasted_iota(jnp.int32, (D, VTAIL), 1)
    acc = jnp.zeros((D, 128), jnp.float32)
    for b in range(B):
        c = i_ref[b]
        cb = jnp.minimum(c // 128, LAST_FULL) * 128
        pltpu.make_async_copy(
            wt_ref.at[:, pl.ds(pl.multiple_of(cb, 128), 128)],
            bufs_ref.at[b], sems.at[b]
        ).wait()
        t = jnp.tanh(bufs_ref[b].astype(jnp.bfloat16).astype(jnp.float32))
        acc = acc + jnp.where(lane128 == c - cb, t, 0.0)
    pltpu.make_async_copy(
        wt_ref.at[:, pl.ds(TAIL, VTAIL)], tail_ref, sems.at[B]
    ).wait()
    t_tail = jnp.tanh(tail_ref[...].astype(jnp.bfloat16).astype(jnp.float32))
    acc_t = jnp.zeros((D, VTAIL), jnp.float32)
    for b in range(B):
        acc_t = acc_t + jnp.where(lane32 == i_ref[b] - TAIL, t_tail, 0.0)
    out_ref[0, 0] = (jnp.sum(acc) + jnp.sum(acc_t)) * (1.0 / B)


_grid_spec = pltpu.PrefetchScalarGridSpec(
    num_scalar_prefetch=1,
    grid=(1,),
    in_specs=[pl.BlockSpec(memory_space=pl.ANY)],
    out_specs=pl.BlockSpec(memory_space=pltpu.SMEM),
    scratch_shapes=[
        pltpu.VMEM((B, D, 128), jnp.float32),
        pltpu.VMEM((D, VTAIL), jnp.float32),
        pltpu.SemaphoreType.DMA((B + 1,)),
    ],
)

_rebar_fb = pl.pallas_call(
    _body,
    grid_spec=_grid_spec,
    out_shape=jax.ShapeDtypeStruct((1, 1), jnp.float32),
)


def kernel(logits, I, w, llm):
    out = _rebar_fb(I.astype(jnp.int32), w.T)
    return out[0, 0]
